# baseline (speedup 1.0000x reference)
import functools

import jax
import jax.numpy as jnp
from jax import lax
from jax.experimental import pallas as pl
from jax.experimental.pallas import tpu as pltpu
from jax.experimental.pallas import tpu_sc as plsc

N = 10000
D = 128
R = 3
E = 100000
NEG_SLOPE = 0.2

EP = 102400
NCH = EP // 128
P1_ROWS = NCH // 16
EPT = P1_ROWS * 128
HALF = 5120
NODE_T = HALF // 16
BN = 400


def _proj_body(x_ref, w_ref, al_ref, ar_ref, z_ref, elr_ref):
    xb = x_ref[...]
    for r in range(R):
        z = jnp.dot(xb, w_ref[r], preferred_element_type=jnp.float32)
        z_ref[r] = z
        elr_ref[r, 0, 0, :] = jnp.sum(z * al_ref[r][None, :], axis=1)
        elr_ref[R + r, 0, 0, :] = jnp.sum(z * ar_ref[r][None, :], axis=1)


def _project(x, W, attn_l, attn_r):
    nb = N // BN
    z, elr = pl.pallas_call(
        _proj_body,
        grid=(nb,),
        in_specs=[
            pl.BlockSpec((BN, D), lambda i: (i, 0)),
            pl.BlockSpec((R, D, D), lambda i: (0, 0, 0)),
            pl.BlockSpec((R, D), lambda i: (0, 0)),
            pl.BlockSpec((R, D), lambda i: (0, 0)),
        ],
        out_specs=[
            pl.BlockSpec((R, BN, D), lambda i: (0, i, 0)),
            pl.BlockSpec((2 * R, 1, 1, BN), lambda i: (0, i, 0, 0)),
        ],
        out_shape=[
            jax.ShapeDtypeStruct((R, N, D), jnp.float32),
            jax.ShapeDtypeStruct((2 * R, nb, 1, BN), jnp.float32),
        ],
    )(x, W, attn_l, attn_r)
    return z, elr.reshape(2 * R, N)


def _lane_bcast(vec, i):
    idx = jnp.full((16, 1), i, jnp.int32)
    dn = lax.GatherDimensionNumbers(
        offset_dims=(), collapsed_slice_dims=(0,), start_index_map=(0,))
    return lax.gather(vec, idx, dn, (1,),
                      mode=lax.GatherScatterMode.PROMISE_IN_BOUNDS)


def _mp_body(z0, z1, z2, el0, el1, el2, er0, er1, er2,
             s0, d0, s1, d1, s2, d2, bias, out,
             el_v, er_v, den_v, src_v, dst_v,
             csrc_v, cdst_v, cee_v, rows_v,
             accrow_v, pv_v, denall_v, zbuf_v, bias_v, acc_sh, den_sh,
             sem0):
    zs = (z0, z1, z2)
    els = (el0, el1, el2)
    ers = (er0, er1, er2)
    srcs = (s0, s1, s2)
    dsts = (d0, d1, d2)
    c = lax.axis_index("c")
    s = lax.axis_index("s")
    zero16 = jnp.zeros((16,), jnp.float32)
    zero16i = jnp.zeros((16,), jnp.int32)
    lo = c * HALF

    for i in range(15):
        el_v[pl.ds(N + i * 16, 16)] = zero16
        er_v[pl.ds(N + i * 16, 16)] = zero16
    for i in range(16):
        for fk in range(8):
            zbuf_v[i, pl.ds(fk * 16, 16)] = zero16
    pltpu.sync_copy(bias, bias_v)

    node0 = s * NODE_T

    for r in range(R):
        def _zacc(kk, carry):
            pltpu.sync_copy(zbuf_v, acc_sh.at[pl.ds(node0 + kk * 16, 16)])
            return carry
        lax.fori_loop(0, NODE_T // 16, _zacc, 0)

        def _zden(i, carry):
            den_v[pl.ds(i * 16, 16)] = zero16
            return carry
        lax.fori_loop(0, HALF // 16, _zden, 0)

        with jax.named_scope("p0_load"):
            pltpu.sync_copy(els[r], el_v.at[pl.ds(0, N)])
            pltpu.sync_copy(ers[r], er_v.at[pl.ds(0, N)])
            pltpu.sync_copy(srcs[r].at[s, 0], src_v)
            pltpu.sync_copy(dsts[r].at[s], dst_v)

        def _p1(g, off):
            s16 = src_v[pl.ds(g * 16, 16)]
            row = lax.shift_right_logical(g, 3)
            col = lax.mul(lax.rem(g, 8), 16)
            d16 = dst_v[row, pl.ds(col, 16)]
            e = plsc.load_gather(el_v, [s16]) + plsc.load_gather(er_v, [d16])
            e = jnp.where(e > 0, e, NEG_SLOPE * e)
            ee = jnp.exp(e)
            dloc = d16 - lo
            mask = (d16 >= lo) & (d16 < lo + HALF) & (d16 < N)
            dloc = jnp.where(mask, dloc, 0)
            plsc.addupdate_scatter(den_v, [dloc], ee, mask=mask)
            plsc.store_compressed(csrc_v.at[pl.ds(off, 16)], s16, mask=mask)
            plsc.store_compressed(cdst_v.at[pl.ds(off, 16)], dloc, mask=mask)
            plsc.store_compressed(cee_v.at[pl.ds(off, 16)], ee, mask=mask)
            return off + jnp.sum(mask.astype(jnp.int32))
        with jax.named_scope("p1_scan"):
            cnt = lax.fori_loop(0, EPT // 16, _p1, jnp.int32(0))
        for k in range(8):
            csrc_v[pl.ds(cnt + k * 16, 16)] = zero16i
            cdst_v[pl.ds(cnt + k * 16, 16)] = zero16i
            cee_v[pl.ds(cnt + k * 16, 16)] = zero16
        nch = lax.shift_right_logical(cnt + 127, 7)

        @pl.when(s < 8)
        def _():
            pltpu.sync_copy(den_v, den_sh.at[pl.ds(s * HALF, HALF)])
        plsc.subcore_barrier()

        def _p2(j, carry):
            base = j * 128
            pltpu.async_copy(zs[r].at[csrc_v.at[pl.ds(base, 128)]],
                             rows_v, sem0).wait()

            def _grp(ib, carry2):
                w_all = cee_v[pl.ds(base + ib * 16, 16)]
                for i2 in range(16):
                    w = _lane_bcast(w_all, i2)
                    i = ib * 16 + i2
                    for fk in range(8):
                        sl = pl.ds(fk * 16, 16)
                        rows_v[i, sl] = rows_v[i, sl] * w
                return carry2
            lax.fori_loop(0, 8, _grp, 0)
            pltpu.sync_copy(rows_v, acc_sh.at[cdst_v.at[pl.ds(base, 128)]],
                            add=True)
            return carry
        with jax.named_scope("p2_rows"):
            lax.fori_loop(0, nch, _p2, 0)
        plsc.subcore_barrier()

        @pl.when(s >= 8)
        def _():
            pltpu.sync_copy(den_sh.at[pl.ds((s - 8) * HALF, HALF)], denall_v)

            def _fold(i, carry):
                sl = pl.ds(i * 16, 16)
                den_v[sl] = den_v[sl] + denall_v[sl]
                return carry
            lax.fori_loop(0, HALF // 16, _fold, 0)
            pltpu.sync_copy(den_v, den_sh.at[pl.ds((s - 8) * HALF, HALF)])
        plsc.subcore_barrier()

        for t in range(8):
            pltpu.sync_copy(den_sh.at[pl.ds(t * HALF + node0, NODE_T)],
                            denall_v.at[pl.ds(t * NODE_T, NODE_T)])

        def _p3(kk, carry):
            lrow = kk * 16
            grow = lo + node0 + lrow

            @pl.when(grow < N)
            def _():
                d16 = denall_v[pl.ds(lrow, 16)]
                for t in range(1, 8):
                    d16 = d16 + denall_v[pl.ds(t * NODE_T + lrow, 16)]
                inv = jnp.where(d16 > 0, 1.0 / jnp.where(d16 > 0, d16, 1.0), 0.0)
                pltpu.sync_copy(acc_sh.at[pl.ds(node0 + lrow, 16)], accrow_v)
                if r > 0:
                    pltpu.sync_copy(out.at[pl.ds(grow, 16)], pv_v)

                def _row(i2, carry2):
                    w = _lane_bcast(inv, i2)
                    for fk in range(8):
                        sl = pl.ds(fk * 16, 16)
                        v = accrow_v[i2, sl] * w
                        if r > 0:
                            v = v + pv_v[i2, sl]
                        if r == R - 1:
                            v = v + bias_v[sl]
                        accrow_v[i2, sl] = v
                    return carry2
                lax.fori_loop(0, 16, _row, 0)
                pltpu.sync_copy(accrow_v, out.at[pl.ds(grow, 16)])
            return carry
        with jax.named_scope("p3_div"):
            lax.fori_loop(0, NODE_T // 16, _p3, 0)
        plsc.subcore_barrier()


_mp_kernel = functools.partial(
    pl.kernel,
    mesh=plsc.VectorSubcoreMesh(core_axis_name="c", subcore_axis_name="s"),
    out_type=jax.ShapeDtypeStruct((N, D), jnp.float32),
    compiler_params=pltpu.CompilerParams(needs_layout_passes=False),
    scratch_types=[
        pltpu.VMEM((N + 240,), jnp.float32),
        pltpu.VMEM((N + 240,), jnp.float32),
        pltpu.VMEM((HALF,), jnp.float32),
        pltpu.VMEM((EPT,), jnp.int32),
        pltpu.VMEM((P1_ROWS, 128), jnp.int32),
        pltpu.VMEM((EPT + 128,), jnp.int32),
        pltpu.VMEM((EPT + 128,), jnp.int32),
        pltpu.VMEM((EPT + 128,), jnp.float32),
        pltpu.VMEM((128, D), jnp.float32),
        pltpu.VMEM((16, D), jnp.float32),
        pltpu.VMEM((16, D), jnp.float32),
        pltpu.VMEM((HALF,), jnp.float32),
        pltpu.VMEM((16, D), jnp.float32),
        pltpu.VMEM((D,), jnp.float32),
        pltpu.VMEM_SHARED((HALF, D), jnp.float32),
        pltpu.VMEM_SHARED((8 * HALF,), jnp.float32),
        pltpu.SemaphoreType.DMA,
    ],
)(_mp_body)


def _prep_edges(ei):
    pad = EP - E
    src = jnp.concatenate(
        [ei[0].astype(jnp.int32), jnp.zeros((pad,), jnp.int32)]
    ).reshape(16, 1, EPT)
    dst = jnp.concatenate(
        [ei[1].astype(jnp.int32), jnp.full((pad,), N + 1, jnp.int32)]
    ).reshape(16, P1_ROWS, 128)
    return src, dst


def kernel(x, edge_index_r0, edge_index_r1, edge_index_r2, W, attn_l, attn_r, h_bias):
    z, elr = _project(x, W, attn_l, attn_r)
    s0, d0 = _prep_edges(edge_index_r0)
    s1, d1 = _prep_edges(edge_index_r1)
    s2, d2 = _prep_edges(edge_index_r2)
    return _mp_kernel(z[0], z[1], z[2],
                      elr[0], elr[1], elr[2], elr[3], elr[4], elr[5],
                      s0, d0, s1, d1, s2, d2, h_bias)


# speedup vs baseline: 1.3916x; 1.3916x over previous
import functools

import jax
import jax.numpy as jnp
from jax import lax
from jax.experimental import pallas as pl
from jax.experimental.pallas import tpu as pltpu
from jax.experimental.pallas import tpu_sc as plsc

N = 10000
D = 128
R = 3
E = 100000
NEG_SLOPE = 0.2

EP = 102400
NCH = EP // 128
P1_ROWS = NCH // 16
EPT = P1_ROWS * 128
HALF = 5120
NODE_T = HALF // 16
BN = 400


def _proj_body(x_ref, w_ref, al_ref, ar_ref, z_ref, elr_ref):
    xb = x_ref[...]
    for r in range(R):
        z = jnp.dot(xb, w_ref[r], preferred_element_type=jnp.float32)
        z_ref[r] = z
        elr_ref[r, 0, 0, :] = jnp.sum(z * al_ref[r][None, :], axis=1)
        elr_ref[R + r, 0, 0, :] = jnp.sum(z * ar_ref[r][None, :], axis=1)


def _project(x, W, attn_l, attn_r):
    nb = N // BN
    z, elr = pl.pallas_call(
        _proj_body,
        grid=(nb,),
        in_specs=[
            pl.BlockSpec((BN, D), lambda i: (i, 0)),
            pl.BlockSpec((R, D, D), lambda i: (0, 0, 0)),
            pl.BlockSpec((R, D), lambda i: (0, 0)),
            pl.BlockSpec((R, D), lambda i: (0, 0)),
        ],
        out_specs=[
            pl.BlockSpec((R, BN, D), lambda i: (0, i, 0)),
            pl.BlockSpec((2 * R, 1, 1, BN), lambda i: (0, i, 0, 0)),
        ],
        out_shape=[
            jax.ShapeDtypeStruct((R, N, D), jnp.float32),
            jax.ShapeDtypeStruct((2 * R, nb, 1, BN), jnp.float32),
        ],
    )(x, W, attn_l, attn_r)
    return z, elr.reshape(2 * R, N)


def _lane_bcast(vec, i):
    idx = jnp.full((16, 1), i, jnp.int32)
    dn = lax.GatherDimensionNumbers(
        offset_dims=(), collapsed_slice_dims=(0,), start_index_map=(0,))
    return lax.gather(vec, idx, dn, (1,),
                      mode=lax.GatherScatterMode.PROMISE_IN_BOUNDS)


def _mp_body(z0, z1, z2, el0, el1, el2, er0, er1, er2,
             s0, d0, s1, d1, s2, d2, bias, out,
             el_v, er_v, den_v, src_v, dst_v,
             csrc_v, cdst_v, cee_v, rows_v,
             accrow_v, pv_v, denall_v, zbuf_v, bias_v, acc_sh, den_sh,
             sem0, sem1, sem2):
    zs = (z0, z1, z2)
    els = (el0, el1, el2)
    ers = (er0, er1, er2)
    srcs = (s0, s1, s2)
    dsts = (d0, d1, d2)
    c = lax.axis_index("c")
    s = lax.axis_index("s")
    zero16 = jnp.zeros((16,), jnp.float32)
    zero16i = jnp.zeros((16,), jnp.int32)
    lo = c * HALF

    for i in range(15):
        el_v[pl.ds(N + i * 16, 16)] = zero16
        er_v[pl.ds(N + i * 16, 16)] = zero16
    for i in range(16):
        for fk in range(8):
            zbuf_v[i, pl.ds(fk * 16, 16)] = zero16
    pltpu.sync_copy(bias, bias_v)

    node0 = s * NODE_T

    for r in range(R):
        def _zacc(kk, carry):
            pltpu.sync_copy(zbuf_v, acc_sh.at[pl.ds(node0 + kk * 16, 16)])
            return carry
        lax.fori_loop(0, NODE_T // 16, _zacc, 0)

        def _zden(i, carry):
            den_v[pl.ds(i * 16, 16)] = zero16
            return carry
        lax.fori_loop(0, HALF // 16, _zden, 0)

        with jax.named_scope("p0_load"):
            pltpu.sync_copy(els[r], el_v.at[pl.ds(0, N)])
            pltpu.sync_copy(ers[r], er_v.at[pl.ds(0, N)])
            pltpu.sync_copy(srcs[r].at[s, 0], src_v)
            pltpu.sync_copy(dsts[r].at[s], dst_v)

        def _p1(g, off):
            s16 = src_v[pl.ds(g * 16, 16)]
            row = lax.shift_right_logical(g, 3)
            col = lax.mul(lax.rem(g, 8), 16)
            d16 = dst_v[row, pl.ds(col, 16)]
            e = plsc.load_gather(el_v, [s16]) + plsc.load_gather(er_v, [d16])
            e = jnp.where(e > 0, e, NEG_SLOPE * e)
            ee = jnp.exp(e)
            dloc = d16 - lo
            mask = (d16 >= lo) & (d16 < lo + HALF) & (d16 < N)
            dloc = jnp.where(mask, dloc, 0)
            plsc.addupdate_scatter(den_v, [dloc], ee, mask=mask)
            plsc.store_compressed(csrc_v.at[pl.ds(off, 16)], s16, mask=mask)
            plsc.store_compressed(cdst_v.at[pl.ds(off, 16)], dloc, mask=mask)
            plsc.store_compressed(cee_v.at[pl.ds(off, 16)], ee, mask=mask)
            return off + jnp.sum(mask.astype(jnp.int32))
        with jax.named_scope("p1_scan"):
            cnt = lax.fori_loop(0, EPT // 16, _p1, jnp.int32(0))
        for k in range(4):
            csrc_v[pl.ds(cnt + k * 16, 16)] = zero16i
            cdst_v[pl.ds(cnt + k * 16, 16)] = zero16i
            cee_v[pl.ds(cnt + k * 16, 16)] = zero16
        nch = lax.shift_right_logical(cnt + 63, 6)

        @pl.when(s < 8)
        def _():
            pltpu.sync_copy(den_v, den_sh.at[pl.ds(s * HALF, HALF)])
        plsc.subcore_barrier()

        CH = 64

        def _gather(j, boff, sg):
            return pltpu.async_copy(
                zs[r].at[csrc_v.at[pl.ds(j * CH, CH)]],
                rows_v.at[pl.ds(boff, CH)], sg)

        def _gather_wait(j, boff, sg):
            pltpu.make_async_copy(
                zs[r].at[csrc_v.at[pl.ds(j * CH, CH)]],
                rows_v.at[pl.ds(boff, CH)], sg).wait()

        def _scatter(j, boff):
            return pltpu.async_copy(
                rows_v.at[pl.ds(boff, CH)],
                acc_sh.at[cdst_v.at[pl.ds(j * CH, CH)]], sem2, add=True)

        def _scatter_wait(j, boff):
            pltpu.make_async_copy(
                rows_v.at[pl.ds(boff, CH)],
                acc_sh.at[cdst_v.at[pl.ds(j * CH, CH)]], sem2).wait()

        def _scale(j, boff):
            base = j * CH

            def _grp(ib, carry2):
                w_all = cee_v[pl.ds(base + ib * 16, 16)]
                for i2 in range(16):
                    w = _lane_bcast(w_all, i2)
                    i = boff + ib * 16 + i2
                    for fk in range(8):
                        sl = pl.ds(fk * 16, 16)
                        rows_v[i, sl] = rows_v[i, sl] * w
                return carry2
            lax.fori_loop(0, CH // 16, _grp, 0)

        gsems = (sem0, sem1)

        with jax.named_scope("p2_rows"):
            @pl.when(0 < nch)
            def _():
                _gather(0, 0, sem0)

            def _p2pair(jj, carry):
                for b in range(2):
                    j = 2 * jj + b

                    @pl.when(j < nch)
                    def _(j=j, b=b):
                        @pl.when(j >= 1)
                        def _():
                            _scatter_wait(j - 1, (1 - b) * CH)

                        @pl.when(j + 1 < nch)
                        def _():
                            _gather(j + 1, (1 - b) * CH, gsems[1 - b])
                        _gather_wait(j, b * CH, gsems[b])
                        _scale(j, b * CH)
                        _scatter(j, b * CH)
                return carry
            lax.fori_loop(0, lax.shift_right_logical(nch + 1, 1), _p2pair, 0)

            @pl.when(nch >= 1)
            def _():
                _scatter_wait(nch - 1, lax.mul(lax.rem(nch + 1, 2), CH))
        plsc.subcore_barrier()

        @pl.when(s >= 8)
        def _():
            pltpu.sync_copy(den_sh.at[pl.ds((s - 8) * HALF, HALF)], denall_v)

            def _fold(i, carry):
                sl = pl.ds(i * 16, 16)
                den_v[sl] = den_v[sl] + denall_v[sl]
                return carry
            lax.fori_loop(0, HALF // 16, _fold, 0)
            pltpu.sync_copy(den_v, den_sh.at[pl.ds((s - 8) * HALF, HALF)])
        plsc.subcore_barrier()

        for t in range(8):
            pltpu.sync_copy(den_sh.at[pl.ds(t * HALF + node0, NODE_T)],
                            denall_v.at[pl.ds(t * NODE_T, NODE_T)])

        def _p3(kk, carry):
            lrow = kk * 16
            grow = lo + node0 + lrow

            @pl.when(grow < N)
            def _():
                d16 = denall_v[pl.ds(lrow, 16)]
                for t in range(1, 8):
                    d16 = d16 + denall_v[pl.ds(t * NODE_T + lrow, 16)]
                inv = jnp.where(d16 > 0, 1.0 / jnp.where(d16 > 0, d16, 1.0), 0.0)
                pltpu.sync_copy(acc_sh.at[pl.ds(node0 + lrow, 16)], accrow_v)
                if r > 0:
                    pltpu.sync_copy(out.at[pl.ds(grow, 16)], pv_v)

                def _row(i2, carry2):
                    w = _lane_bcast(inv, i2)
                    for fk in range(8):
                        sl = pl.ds(fk * 16, 16)
                        v = accrow_v[i2, sl] * w
                        if r > 0:
                            v = v + pv_v[i2, sl]
                        if r == R - 1:
                            v = v + bias_v[sl]
                        accrow_v[i2, sl] = v
                    return carry2
                lax.fori_loop(0, 16, _row, 0)
                pltpu.sync_copy(accrow_v, out.at[pl.ds(grow, 16)])
            return carry
        with jax.named_scope("p3_div"):
            lax.fori_loop(0, NODE_T // 16, _p3, 0)
        plsc.subcore_barrier()


_mp_kernel = functools.partial(
    pl.kernel,
    mesh=plsc.VectorSubcoreMesh(core_axis_name="c", subcore_axis_name="s"),
    out_type=jax.ShapeDtypeStruct((N, D), jnp.float32),
    compiler_params=pltpu.CompilerParams(needs_layout_passes=False),
    scratch_types=[
        pltpu.VMEM((N + 240,), jnp.float32),
        pltpu.VMEM((N + 240,), jnp.float32),
        pltpu.VMEM((HALF,), jnp.float32),
        pltpu.VMEM((EPT,), jnp.int32),
        pltpu.VMEM((P1_ROWS, 128), jnp.int32),
        pltpu.VMEM((EPT + 128,), jnp.int32),
        pltpu.VMEM((EPT + 128,), jnp.int32),
        pltpu.VMEM((EPT + 128,), jnp.float32),
        pltpu.VMEM((128, D), jnp.float32),
        pltpu.VMEM((16, D), jnp.float32),
        pltpu.VMEM((16, D), jnp.float32),
        pltpu.VMEM((HALF,), jnp.float32),
        pltpu.VMEM((16, D), jnp.float32),
        pltpu.VMEM((D,), jnp.float32),
        pltpu.VMEM_SHARED((HALF, D), jnp.float32),
        pltpu.VMEM_SHARED((8 * HALF,), jnp.float32),
        pltpu.SemaphoreType.DMA,
        pltpu.SemaphoreType.DMA,
        pltpu.SemaphoreType.DMA,
    ],
)(_mp_body)


def _prep_edges(ei):
    pad = EP - E
    src = jnp.concatenate(
        [ei[0].astype(jnp.int32), jnp.zeros((pad,), jnp.int32)]
    ).reshape(16, 1, EPT)
    dst = jnp.concatenate(
        [ei[1].astype(jnp.int32), jnp.full((pad,), N + 1, jnp.int32)]
    ).reshape(16, P1_ROWS, 128)
    return src, dst


def kernel(x, edge_index_r0, edge_index_r1, edge_index_r2, W, attn_l, attn_r, h_bias):
    z, elr = _project(x, W, attn_l, attn_r)
    s0, d0 = _prep_edges(edge_index_r0)
    s1, d1 = _prep_edges(edge_index_r1)
    s2, d2 = _prep_edges(edge_index_r2)
    return _mp_kernel(z[0], z[1], z[2],
                      elr[0], elr[1], elr[2], elr[3], elr[4], elr[5],
                      s0, d0, s1, d1, s2, d2, h_bias)
